# baseline (speedup 1.0000x reference)
import functools

import jax
import jax.numpy as jnp
from jax import lax
from jax.experimental import pallas as pl
from jax.experimental.pallas import tpu as pltpu
from jax.experimental.pallas import tpu_sc as plsc

_B = 16384
_VOCAB_F1 = 3
_VOCAB_F2 = 11
_EMB_DIM = 10
_H1 = 20
_NCOMBO = _VOCAB_F1 * _VOCAB_F2
_TBL = 64

_NC, _NS, _L = 2, 16, 16
_NW = _NC * _NS
_BPW = _B // _NW


def _table_body(emb_ref, w1_ref, b1_ref, w2_ref, b2_ref, out_ref):
    c = lax.broadcasted_iota(jnp.int32, (_TBL, 1), 0)
    a = c // _VOCAB_F2
    b = c % _VOCAB_F2
    oh1 = (a == lax.broadcasted_iota(jnp.int32, (_TBL, _VOCAB_F1), 1)).astype(jnp.float32)
    oh2 = (b == lax.broadcasted_iota(jnp.int32, (_TBL, _VOCAB_F2), 1)).astype(jnp.float32)
    emb = jnp.dot(oh2, emb_ref[...], preferred_element_type=jnp.float32)
    h = (jnp.dot(oh1, w1_ref[: _VOCAB_F1, :], preferred_element_type=jnp.float32)
         + jnp.dot(emb, w1_ref[_VOCAB_F1:, :], preferred_element_type=jnp.float32)
         + jnp.broadcast_to(b1_ref[...], (_TBL, _H1)))
    h = jnp.maximum(h, 0.0)
    y = lax.dot_general(w2_ref[...], h, (((0,), (1,)), ((), ())),
                        preferred_element_type=jnp.float32)
    y = y + jnp.broadcast_to(b2_ref[...], (1, _TBL))
    out_ref[...] = y / (1.0 + jnp.abs(y))


_table_call = pl.pallas_call(
    _table_body,
    out_shape=jax.ShapeDtypeStruct((1, _TBL), jnp.float32),
)


@functools.partial(
    pl.kernel,
    out_type=jax.ShapeDtypeStruct((_B,), jnp.float32),
    mesh=plsc.VectorSubcoreMesh(core_axis_name="c", subcore_axis_name="s"),
    compiler_params=pltpu.CompilerParams(needs_layout_passes=False),
    scratch_types=[
        pltpu.VMEM((_BPW,), jnp.int32),
        pltpu.VMEM((_BPW,), jnp.int32),
        pltpu.VMEM((_TBL,), jnp.float32),
        pltpu.VMEM((_BPW,), jnp.float32),
        pltpu.SemaphoreType.DMA,
    ],
)
def _sc_gather(f1_hbm, f2_hbm, tbl_hbm, out_hbm, f1_v, f2_v, tbl_v, out_v, sem):
    wid = lax.axis_index("s") * _NC + lax.axis_index("c")
    base = wid * _BPW
    c1 = pltpu.async_copy(tbl_hbm, tbl_v, sem)
    c2 = pltpu.async_copy(f1_hbm.at[pl.ds(base, _BPW)], f1_v, sem)
    c3 = pltpu.async_copy(f2_hbm.at[pl.ds(base, _BPW)], f2_v, sem)
    c1.wait()
    c2.wait()
    c3.wait()
    half = _BPW // 2
    for i in range(half // _L):
        s = pl.ds(i * _L, _L)
        idx = f1_v[s] * _VOCAB_F2 + f2_v[s]
        out_v[s] = plsc.load_gather(tbl_v, [idx])
    o1 = pltpu.async_copy(out_v.at[pl.ds(0, half)],
                          out_hbm.at[pl.ds(base, half)], sem)
    for i in range(half // _L, _BPW // _L):
        s = pl.ds(i * _L, _L)
        idx = f1_v[s] * _VOCAB_F2 + f2_v[s]
        out_v[s] = plsc.load_gather(tbl_v, [idx])
    o2 = pltpu.async_copy(out_v.at[pl.ds(half, half)],
                          out_hbm.at[pl.ds(base + half, half)], sem)
    o1.wait()
    o2.wait()


def kernel(f1, f2, emb_f2, W1, b1, W2, b2):
    f1 = f1.astype(jnp.int32)
    f2 = f2.astype(jnp.int32)
    tbl = _table_call(emb_f2, W1, b1, W2, b2)
    out = _sc_gather(f1, f2, tbl.reshape(_TBL))
    return out.reshape(_B, 1)
